# baseline (speedup 1.0000x reference)
import functools

import jax
import jax.numpy as jnp
from jax import lax
from jax.experimental import pallas as pl
from jax.experimental.pallas import tpu as pltpu
from jax.experimental.pallas import tpu_sc as plsc

_ALPHA = 0.1
_LANES = 16
_SC_PARAMS = pltpu.CompilerParams(needs_layout_passes=False)


def _scores_body(x_ref, w_ref, z_ref, s_ref):
    xb = x_ref[...].astype(jnp.bfloat16)
    wb = w_ref[...].astype(jnp.bfloat16)
    z = jax.lax.dot_general(xb, wb.T, (((1,), (0,)), ((), ())),
                            preferred_element_type=jnp.float32)
    z = z.reshape(z_ref.shape)
    z_ref[...] = z
    s_ref[...] = _ALPHA * (1.0 / (1.0 + jnp.exp(-z)))


def _compute_scores(x2d, w):
    n, h = x2d.shape
    rows = 1024
    return pl.pallas_call(
        _scores_body,
        grid=(n // rows,),
        in_specs=[
            pl.BlockSpec((rows, h), lambda i: (i, 0)),
            pl.BlockSpec((1, h), lambda i: (0, 0)),
        ],
        out_specs=[pl.BlockSpec((rows // 128, 128), lambda i: (i, 0))] * 2,
        out_shape=[jax.ShapeDtypeStruct((n // 128, 128), jnp.float32)] * 2,
    )(x2d, w)


def _thresh_body(k, b, s, z_ref, t_ref, n_ref):
    z = z_ref[...].reshape(b, s)
    u = lax.bitcast_convert_type(z, jnp.int32)
    key = jnp.bitwise_xor(
        u, jnp.bitwise_and(jnp.right_shift(u, 31), jnp.int32(0x7FFFFFFF)))

    cnt_nn = jnp.sum((key >= 0).astype(jnp.int32), axis=1, keepdims=True)
    t0 = jnp.where(cnt_nn >= k, jnp.int32(0), jnp.int32(-2147483648))

    def step(i, t_acc):
        bit = 30 - i
        cand = jnp.bitwise_or(t_acc, jnp.left_shift(jnp.int32(1), bit))
        cnt = jnp.sum((key >= cand).astype(jnp.int32), axis=1, keepdims=True)
        return jnp.where(cnt >= k, cand, t_acc)

    tk = lax.fori_loop(0, 31, step, t0)
    uz = jnp.bitwise_xor(
        tk, jnp.bitwise_and(jnp.right_shift(tk, 31), jnp.int32(0x7FFFFFFF)))
    t_z = lax.bitcast_convert_type(uz, jnp.float32)
    n_gt = jnp.sum((z > t_z).astype(jnp.int32), axis=1, keepdims=True)
    need = k - n_gt
    t_ref[...] = jnp.broadcast_to(t_z, (b, _LANES))
    n_ref[...] = jnp.broadcast_to(need, (b, _LANES))


def _compute_threshold(z2d, b, s, k):
    return pl.pallas_call(
        functools.partial(_thresh_body, k, b, s),
        in_specs=[pl.BlockSpec(z2d.shape, lambda: (0, 0))],
        out_specs=[
            pl.BlockSpec((b, _LANES), lambda: (0, 0)),
            pl.BlockSpec((b, _LANES), lambda: (0, 0)),
        ],
        out_shape=[
            jax.ShapeDtypeStruct((b, _LANES), jnp.float32),
            jax.ShapeDtypeStruct((b, _LANES), jnp.int32),
        ],
    )(z2d)


def _compact(z_flat, s_flat, t_flat, n_flat, b, s, k):
    mesh = plsc.VectorSubcoreMesh(core_axis_name="c", subcore_axis_name="s",
                                  num_cores=2, num_subcores=16)

    @functools.partial(
        pl.kernel,
        out_type=(
            jax.ShapeDtypeStruct((b * k,), jnp.int32),
            jax.ShapeDtypeStruct((b * k,), jnp.float32),
        ),
        mesh=mesh,
        scratch_types=[
            pltpu.VMEM((s,), jnp.float32),
            pltpu.VMEM((s,), jnp.float32),
            pltpu.VMEM((_LANES,), jnp.float32),
            pltpu.VMEM((_LANES,), jnp.int32),
            pltpu.VMEM((k + _LANES,), jnp.int32),
            pltpu.VMEM((k + _LANES,), jnp.float32),
        ],
        compiler_params=_SC_PARAMS,
    )
    def _compact_kernel(z_hbm, s_hbm, t_hbm, n_hbm, gi_hbm, ws_hbm,
                        zv, sv, tv, nv, gbuf, wbuf):
        cid = lax.axis_index("c")
        sid = lax.axis_index("s")

        @pl.when(jnp.logical_and(cid == 0, sid < b))
        def _():
            bb = sid
            pltpu.sync_copy(z_hbm.at[pl.ds(bb * s, s)], zv)
            pltpu.sync_copy(s_hbm.at[pl.ds(bb * s, s)], sv)
            pltpu.sync_copy(t_hbm.at[pl.ds(bb * _LANES, _LANES)], tv)
            pltpu.sync_copy(n_hbm.at[pl.ds(bb * _LANES, _LANES)], nv)
            t = tv[...]
            nd = nv[...]
            base = bb * s

            def step(j, carry):
                off, eqc = carry
                zv16 = zv[pl.ds(j * _LANES, _LANES)]
                sv16 = sv[pl.ds(j * _LANES, _LANES)]
                idx = lax.iota(jnp.int32, _LANES) + (base + j * _LANES)
                gt = zv16 > t
                eqm = zv16 == t
                eqi = eqm.astype(jnp.int32)
                csum = plsc.cumsum(eqi)
                rank_before = (csum - eqi) + eqc
                sel = jnp.logical_or(gt, jnp.logical_and(eqm, rank_before < nd))
                plsc.store_compressed(gbuf.at[pl.ds(off, _LANES)], idx,
                                      mask=sel)
                plsc.store_compressed(wbuf.at[pl.ds(off, _LANES)], sv16,
                                      mask=sel)
                off = off + jnp.sum(sel.astype(jnp.int32))
                eqc = eqc + jnp.sum(eqi)
                return off, eqc

            lax.fori_loop(0, s // _LANES, step,
                          (jnp.int32(0), jnp.int32(0)))
            pltpu.sync_copy(gbuf.at[pl.ds(0, k)], gi_hbm.at[pl.ds(bb * k, k)])
            pltpu.sync_copy(wbuf.at[pl.ds(0, k)], ws_hbm.at[pl.ds(bb * k, k)])

    return _compact_kernel(z_flat, s_flat, t_flat, n_flat)


def _gather_scale(x2d, gidx, wsel, b, s, h, k):
    mesh = plsc.VectorSubcoreMesh(core_axis_name="c", subcore_axis_name="s",
                                  num_cores=2, num_subcores=16)
    nw = 32
    rpw = (b * k) // nw
    ch = 16

    @functools.partial(
        pl.kernel,
        out_type=jax.ShapeDtypeStruct((b * k, h), jnp.float32),
        mesh=mesh,
        scratch_types=[
            pltpu.VMEM((rpw,), jnp.int32),
            pltpu.VMEM((rpw,), jnp.float32),
            pltpu.VMEM((ch, h), jnp.float32),
            pltpu.SemaphoreType.DMA,
            pltpu.SemaphoreType.DMA,
        ],
        compiler_params=_SC_PARAMS,
    )
    def _gather_kernel(x_hbm, gi_hbm, ws_hbm, o_hbm, gi, wv, buf, gsem, osem):
        cid = lax.axis_index("c")
        sid = lax.axis_index("s")
        wid = sid * 2 + cid
        row0 = wid * rpw
        pltpu.sync_copy(gi_hbm.at[pl.ds(row0, rpw)], gi)
        pltpu.sync_copy(ws_hbm.at[pl.ds(row0, rpw)], wv)

        @pl.loop(0, rpw // ch)
        def _(c):
            pltpu.async_copy(x_hbm.at[gi.at[pl.ds(c * ch, ch)]], buf,
                             gsem).wait()
            for r in range(ch):
                wr = plsc.load_gather(
                    wv, [jnp.full((_LANES,), c * ch + r, jnp.int32)])

                @pl.loop(0, h, step=_LANES)
                def _(l):
                    buf[r, pl.ds(l, _LANES)] = buf[r, pl.ds(l, _LANES)] * wr

            pltpu.async_copy(buf, o_hbm.at[pl.ds(row0 + c * ch, ch)],
                             osem).wait()

    return _gather_kernel(x2d, gidx, wsel)


def kernel(x, W_router):
    b, s, h = x.shape
    k = max(1, (s * 1) // 2)
    x2d = x.reshape(b * s, h)
    z2d, s2d = _compute_scores(x2d, W_router)
    t_out, n_out = _compute_threshold(z2d, b, s, k)
    gidx, wsel = _compact(z2d.reshape(b * s), s2d.reshape(b * s),
                          t_out.reshape(-1), n_out.reshape(-1), b, s, k)
    out2d = _gather_scale(x2d, gidx, wsel, b, s, h, k)
    return out2d.reshape(b, k, h)


# speedup vs baseline: 1.7575x; 1.7575x over previous
import functools

import jax
import jax.numpy as jnp
from jax import lax
from jax.experimental import pallas as pl
from jax.experimental.pallas import tpu as pltpu
from jax.experimental.pallas import tpu_sc as plsc

_ALPHA = 0.1
_LANES = 16
_SC_PARAMS = pltpu.CompilerParams(needs_layout_passes=False)


def _scores_body(x_ref, w_ref, z_ref, s_ref):
    xb = x_ref[...].astype(jnp.bfloat16)
    wb = w_ref[...].astype(jnp.bfloat16)
    z = jax.lax.dot_general(xb, wb.T, (((1,), (0,)), ((), ())),
                            preferred_element_type=jnp.float32)
    z = z.reshape(z_ref.shape)
    z_ref[...] = z
    s_ref[...] = _ALPHA * (1.0 / (1.0 + jnp.exp(-z)))


def _compute_scores(x2d, w):
    n, h = x2d.shape
    rows = 1024
    return pl.pallas_call(
        _scores_body,
        grid=(n // rows,),
        in_specs=[
            pl.BlockSpec((rows, h), lambda i: (i, 0)),
            pl.BlockSpec((1, h), lambda i: (0, 0)),
        ],
        out_specs=[pl.BlockSpec((rows // 128, 128), lambda i: (i, 0))] * 2,
        out_shape=[jax.ShapeDtypeStruct((n // 128, 128), jnp.float32)] * 2,
        compiler_params=pltpu.CompilerParams(
            dimension_semantics=("parallel",)),
    )(x2d, w)


def _thresh_body(k, b, s, z_ref, t_ref, n_ref):
    z = z_ref[...].reshape(b, s)
    u = lax.bitcast_convert_type(z, jnp.int32)
    key = jnp.bitwise_xor(
        u, jnp.bitwise_and(jnp.right_shift(u, 31), jnp.int32(0x7FFFFFFF)))

    cnt_nn = jnp.sum((key >= 0).astype(jnp.int32), axis=1, keepdims=True)
    t0 = jnp.where(cnt_nn >= k, jnp.int32(0), jnp.int32(-2147483648))

    def step(i, t_acc):
        bit = 30 - i
        cand = jnp.bitwise_or(t_acc, jnp.left_shift(jnp.int32(1), bit))
        cnt = jnp.sum((key >= cand).astype(jnp.int32), axis=1, keepdims=True)
        return jnp.where(cnt >= k, cand, t_acc)

    tk = lax.fori_loop(0, 31, step, t0)
    uz = jnp.bitwise_xor(
        tk, jnp.bitwise_and(jnp.right_shift(tk, 31), jnp.int32(0x7FFFFFFF)))
    t_z = lax.bitcast_convert_type(uz, jnp.float32)
    n_gt = jnp.sum((z > t_z).astype(jnp.int32), axis=1, keepdims=True)
    need = k - n_gt
    t_ref[...] = jnp.broadcast_to(t_z, (b, _LANES))
    n_ref[...] = jnp.broadcast_to(need, (b, _LANES))


def _compute_threshold(z2d, b, s, k):
    return pl.pallas_call(
        functools.partial(_thresh_body, k, b, s),
        in_specs=[pl.BlockSpec(z2d.shape, lambda: (0, 0))],
        out_specs=[
            pl.BlockSpec((b, _LANES), lambda: (0, 0)),
            pl.BlockSpec((b, _LANES), lambda: (0, 0)),
        ],
        out_shape=[
            jax.ShapeDtypeStruct((b, _LANES), jnp.float32),
            jax.ShapeDtypeStruct((b, _LANES), jnp.int32),
        ],
    )(z2d)


def _compact(z_flat, s_flat, t_flat, n_flat, b, s, k):
    mesh = plsc.VectorSubcoreMesh(core_axis_name="c", subcore_axis_name="s",
                                  num_cores=2, num_subcores=16)

    @functools.partial(
        pl.kernel,
        out_type=(
            jax.ShapeDtypeStruct((b * k,), jnp.int32),
            jax.ShapeDtypeStruct((b * k,), jnp.float32),
        ),
        mesh=mesh,
        scratch_types=[
            pltpu.VMEM((s,), jnp.float32),
            pltpu.VMEM((s,), jnp.float32),
            pltpu.VMEM((_LANES,), jnp.float32),
            pltpu.VMEM((_LANES,), jnp.int32),
            pltpu.VMEM((k + _LANES,), jnp.int32),
            pltpu.VMEM((k + _LANES,), jnp.float32),
        ],
        compiler_params=_SC_PARAMS,
    )
    def _compact_kernel(z_hbm, s_hbm, t_hbm, n_hbm, gi_hbm, ws_hbm,
                        zv, sv, tv, nv, gbuf, wbuf):
        cid = lax.axis_index("c")
        sid = lax.axis_index("s")

        @pl.when(jnp.logical_and(cid == 0, sid < b))
        def _():
            bb = sid
            pltpu.sync_copy(z_hbm.at[pl.ds(bb * s, s)], zv)
            pltpu.sync_copy(s_hbm.at[pl.ds(bb * s, s)], sv)
            pltpu.sync_copy(t_hbm.at[pl.ds(bb * _LANES, _LANES)], tv)
            pltpu.sync_copy(n_hbm.at[pl.ds(bb * _LANES, _LANES)], nv)
            t = tv[...]
            nd = nv[...]
            base = bb * s

            def step(j, carry):
                off, eqc = carry
                zv16 = zv[pl.ds(j * _LANES, _LANES)]
                sv16 = sv[pl.ds(j * _LANES, _LANES)]
                idx = lax.iota(jnp.int32, _LANES) + (base + j * _LANES)
                gt = zv16 > t
                eqm = zv16 == t
                eqi = eqm.astype(jnp.int32)
                csum = plsc.cumsum(eqi)
                rank_before = (csum - eqi) + eqc
                sel = jnp.logical_or(gt, jnp.logical_and(eqm, rank_before < nd))
                plsc.store_compressed(gbuf.at[pl.ds(off, _LANES)], idx,
                                      mask=sel)
                plsc.store_compressed(wbuf.at[pl.ds(off, _LANES)], sv16,
                                      mask=sel)
                off = off + jnp.sum(sel.astype(jnp.int32))
                eqc = eqc + jnp.sum(eqi)
                return off, eqc

            lax.fori_loop(0, s // _LANES, step,
                          (jnp.int32(0), jnp.int32(0)))
            pltpu.sync_copy(gbuf.at[pl.ds(0, k)], gi_hbm.at[pl.ds(bb * k, k)])
            pltpu.sync_copy(wbuf.at[pl.ds(0, k)], ws_hbm.at[pl.ds(bb * k, k)])

    return _compact_kernel(z_flat, s_flat, t_flat, n_flat)


def _gather(x2d, gidx, b, s, h, k):
    mesh = plsc.VectorSubcoreMesh(core_axis_name="c", subcore_axis_name="s",
                                  num_cores=2, num_subcores=16)
    nw = 32
    rpw = (b * k) // nw
    ch = 16
    nch = rpw // ch

    @functools.partial(
        pl.kernel,
        out_type=jax.ShapeDtypeStruct((b * k, h), jnp.float32),
        mesh=mesh,
        scratch_types=[
            pltpu.VMEM((rpw,), jnp.int32),
            pltpu.VMEM((ch, h), jnp.float32),
            pltpu.VMEM((ch, h), jnp.float32),
            pltpu.SemaphoreType.DMA,
            pltpu.SemaphoreType.DMA,
            pltpu.SemaphoreType.DMA,
            pltpu.SemaphoreType.DMA,
        ],
        compiler_params=_SC_PARAMS,
    )
    def _gather_kernel(x_hbm, gi_hbm, o_hbm, gi, bufa, bufb,
                       gsa, gsb, ssa, ssb):
        cid = lax.axis_index("c")
        sid = lax.axis_index("s")
        wid = sid * 2 + cid
        row0 = wid * rpw
        pltpu.sync_copy(gi_hbm.at[pl.ds(row0, rpw)], gi)
        bufs = ((bufa, gsa, ssa), (bufb, gsb, ssb))

        def start_g(c):
            buf, gs, _ = bufs[c % 2]
            return pltpu.async_copy(x_hbm.at[gi.at[pl.ds(c * ch, ch)]],
                                    buf, gs)

        def start_s(c):
            buf, _, ss = bufs[c % 2]
            return pltpu.async_copy(buf, o_hbm.at[pl.ds(row0 + c * ch, ch)],
                                    ss)

        ds_g = [None] * nch
        ds_s = [None] * nch
        ds_g[0] = start_g(0)
        for c in range(nch):
            ds_g[c].wait()
            ds_s[c] = start_s(c)
            if c + 1 < nch:
                if c >= 1:
                    ds_s[c - 1].wait()
                ds_g[c + 1] = start_g(c + 1)
        ds_s[nch - 1].wait()

    return _gather_kernel(x2d, gidx)


def _scale_body(g_ref, w_ref, o_ref):
    o_ref[...] = g_ref[...] * w_ref[...]


def _scale_rows(gath, wcol, rows_blk=1024):
    n, h = gath.shape
    return pl.pallas_call(
        _scale_body,
        grid=(n // rows_blk,),
        in_specs=[
            pl.BlockSpec((rows_blk, h), lambda i: (i, 0)),
            pl.BlockSpec((rows_blk, 1), lambda i: (i, 0)),
        ],
        out_specs=pl.BlockSpec((rows_blk, h), lambda i: (i, 0)),
        out_shape=jax.ShapeDtypeStruct((n, h), jnp.float32),
        compiler_params=pltpu.CompilerParams(
            dimension_semantics=("parallel",)),
    )(gath, wcol)


def kernel(x, W_router):
    b, s, h = x.shape
    k = max(1, (s * 1) // 2)
    x2d = x.reshape(b * s, h)
    z2d, s2d = _compute_scores(x2d, W_router)
    t_out, n_out = _compute_threshold(z2d, b, s, k)
    gidx, wsel = _compact(z2d.reshape(b * s), s2d.reshape(b * s),
                          t_out.reshape(-1), n_out.reshape(-1), b, s, k)
    gath = _gather(x2d, gidx, b, s, h, k)
    out2d = _scale_rows(gath, wsel.reshape(b * k, 1))
    return out2d.reshape(b, k, h)


# speedup vs baseline: 2.3169x; 1.3183x over previous
import functools

import jax
import jax.numpy as jnp
from jax import lax
from jax.experimental import pallas as pl
from jax.experimental.pallas import tpu as pltpu
from jax.experimental.pallas import tpu_sc as plsc

_ALPHA = 0.1
_LANES = 16
_SC_PARAMS = pltpu.CompilerParams(needs_layout_passes=False)


def _scores_body(x_ref, w_ref, z_ref, s_ref):
    xb = x_ref[...].astype(jnp.bfloat16)
    wb = w_ref[...].astype(jnp.bfloat16)
    z = jax.lax.dot_general(xb, wb.T, (((1,), (0,)), ((), ())),
                            preferred_element_type=jnp.float32)
    z = z.reshape(z_ref.shape)
    z_ref[...] = z
    s_ref[...] = _ALPHA * (1.0 / (1.0 + jnp.exp(-z)))


def _compute_scores(x2d, w):
    n, h = x2d.shape
    rows = 1024
    return pl.pallas_call(
        _scores_body,
        grid=(n // rows,),
        in_specs=[
            pl.BlockSpec((rows, h), lambda i: (i, 0)),
            pl.BlockSpec((1, h), lambda i: (0, 0)),
        ],
        out_specs=[pl.BlockSpec((rows // 128, 128), lambda i: (i, 0))] * 2,
        out_shape=[jax.ShapeDtypeStruct((n // 128, 128), jnp.float32)] * 2,
        compiler_params=pltpu.CompilerParams(
            dimension_semantics=("parallel",)),
    )(x2d, w)


def _thresh_body(k, b, s, z_ref, t_ref, n_ref):
    z = z_ref[...].reshape(b, s)
    u = lax.bitcast_convert_type(z, jnp.int32)
    key = jnp.bitwise_xor(
        u, jnp.bitwise_and(jnp.right_shift(u, 31), jnp.int32(0x7FFFFFFF)))

    cnt_nn = jnp.sum((key >= 0).astype(jnp.int32), axis=1, keepdims=True)
    t0 = jnp.where(cnt_nn >= k, jnp.int32(0), jnp.int32(-2147483648))

    def step(i, t_acc):
        bit = 30 - i
        cand = jnp.bitwise_or(t_acc, jnp.left_shift(jnp.int32(1), bit))
        cnt = jnp.sum((key >= cand).astype(jnp.int32), axis=1, keepdims=True)
        return jnp.where(cnt >= k, cand, t_acc)

    tk = lax.fori_loop(0, 31, step, t0)
    uz = jnp.bitwise_xor(
        tk, jnp.bitwise_and(jnp.right_shift(tk, 31), jnp.int32(0x7FFFFFFF)))
    t_z = lax.bitcast_convert_type(uz, jnp.float32)
    n_gt = jnp.sum((z > t_z).astype(jnp.int32), axis=1, keepdims=True)
    need = k - n_gt
    t_ref[...] = jnp.broadcast_to(t_z, (b, _LANES))
    n_ref[...] = jnp.broadcast_to(need, (b, _LANES))


def _compute_threshold(z2d, b, s, k):
    return pl.pallas_call(
        functools.partial(_thresh_body, k, b, s),
        in_specs=[pl.BlockSpec(z2d.shape, lambda: (0, 0))],
        out_specs=[
            pl.BlockSpec((b, _LANES), lambda: (0, 0)),
            pl.BlockSpec((b, _LANES), lambda: (0, 0)),
        ],
        out_shape=[
            jax.ShapeDtypeStruct((b, _LANES), jnp.float32),
            jax.ShapeDtypeStruct((b, _LANES), jnp.int32),
        ],
    )(z2d)


def _compact(z_flat, s_flat, t_flat, n_flat, b, s, k):
    mesh = plsc.VectorSubcoreMesh(core_axis_name="c", subcore_axis_name="s",
                                  num_cores=2, num_subcores=16)

    @functools.partial(
        pl.kernel,
        out_type=(
            jax.ShapeDtypeStruct((b * k,), jnp.int32),
            jax.ShapeDtypeStruct((b * k,), jnp.float32),
        ),
        mesh=mesh,
        scratch_types=[
            pltpu.VMEM((s,), jnp.float32),
            pltpu.VMEM((s,), jnp.float32),
            pltpu.VMEM((_LANES,), jnp.float32),
            pltpu.VMEM((_LANES,), jnp.int32),
            pltpu.VMEM((k + _LANES,), jnp.int32),
            pltpu.VMEM((k + _LANES,), jnp.float32),
        ],
        compiler_params=_SC_PARAMS,
    )
    def _compact_kernel(z_hbm, s_hbm, t_hbm, n_hbm, gi_hbm, ws_hbm,
                        zv, sv, tv, nv, gbuf, wbuf):
        cid = lax.axis_index("c")
        sid = lax.axis_index("s")

        @pl.when(jnp.logical_and(cid == 0, sid < b))
        def _():
            bb = sid
            pltpu.sync_copy(z_hbm.at[pl.ds(bb * s, s)], zv)
            pltpu.sync_copy(s_hbm.at[pl.ds(bb * s, s)], sv)
            pltpu.sync_copy(t_hbm.at[pl.ds(bb * _LANES, _LANES)], tv)
            pltpu.sync_copy(n_hbm.at[pl.ds(bb * _LANES, _LANES)], nv)
            t = tv[...]
            nd = nv[...]
            base = bb * s

            def step(j, carry):
                off, eqc = carry
                zv16 = zv[pl.ds(j * _LANES, _LANES)]
                sv16 = sv[pl.ds(j * _LANES, _LANES)]
                idx = lax.iota(jnp.int32, _LANES) + (base + j * _LANES)
                gt = zv16 > t
                eqm = zv16 == t
                eqi = eqm.astype(jnp.int32)
                csum = plsc.cumsum(eqi)
                rank_before = (csum - eqi) + eqc
                sel = jnp.logical_or(gt, jnp.logical_and(eqm, rank_before < nd))
                plsc.store_compressed(gbuf.at[pl.ds(off, _LANES)], idx,
                                      mask=sel)
                plsc.store_compressed(wbuf.at[pl.ds(off, _LANES)], sv16,
                                      mask=sel)
                off = off + jnp.sum(sel.astype(jnp.int32))
                eqc = eqc + jnp.sum(eqi)
                return off, eqc

            lax.fori_loop(0, s // _LANES, step,
                          (jnp.int32(0), jnp.int32(0)))
            pltpu.sync_copy(gbuf.at[pl.ds(0, k)], gi_hbm.at[pl.ds(bb * k, k)])
            pltpu.sync_copy(wbuf.at[pl.ds(0, k)], ws_hbm.at[pl.ds(bb * k, k)])

    return _compact_kernel(z_flat, s_flat, t_flat, n_flat)


def _gather(x2d, gidx, wsel, b, s, h, k):
    mesh = plsc.VectorSubcoreMesh(core_axis_name="c", subcore_axis_name="s",
                                  num_cores=2, num_subcores=16)
    nw = 32
    rpw = (b * k) // nw
    ch = 16
    nch = rpw // ch

    @functools.partial(
        pl.kernel,
        out_type=jax.ShapeDtypeStruct((b * k, h), jnp.float32),
        mesh=mesh,
        scratch_types=[
            pltpu.VMEM((rpw,), jnp.int32),
            pltpu.VMEM((rpw,), jnp.float32),
            pltpu.VMEM((ch, h), jnp.float32),
            pltpu.VMEM((ch, h), jnp.float32),
            pltpu.SemaphoreType.DMA,
            pltpu.SemaphoreType.DMA,
            pltpu.SemaphoreType.DMA,
            pltpu.SemaphoreType.DMA,
        ],
        compiler_params=_SC_PARAMS,
    )
    def _gather_kernel(x_hbm, gi_hbm, ws_hbm, o_hbm, gi, wv, bufa, bufb,
                       gsa, gsb, ssa, ssb):
        cid = lax.axis_index("c")
        sid = lax.axis_index("s")
        wid = sid * 2 + cid
        row0 = wid * rpw
        pltpu.sync_copy(gi_hbm.at[pl.ds(row0, rpw)], gi)
        pltpu.sync_copy(ws_hbm.at[pl.ds(row0, rpw)], wv)
        bufs = ((bufa, gsa, ssa), (bufb, gsb, ssb))

        def start_g(c):
            buf, gs, _ = bufs[c % 2]
            return pltpu.async_copy(x_hbm.at[gi.at[pl.ds(c * ch, ch)]],
                                    buf, gs)

        def start_s(c):
            buf, _, ss = bufs[c % 2]
            return pltpu.async_copy(buf, o_hbm.at[pl.ds(row0 + c * ch, ch)],
                                    ss)

        def scale(c):
            buf = bufs[c % 2][0]

            @pl.loop(0, ch)
            def _(r):
                wr = plsc.load_gather(
                    wv, [jnp.full((_LANES,), c * ch + r, jnp.int32)])

                @pl.loop(0, h, step=_LANES, unroll=8)
                def _(l):
                    buf[r, pl.ds(l, _LANES)] = buf[r, pl.ds(l, _LANES)] * wr

        ds_g = [None] * nch
        ds_s = [None] * nch
        ds_g[0] = start_g(0)
        for c in range(nch):
            ds_g[c].wait()
            if c + 1 < nch:
                if c >= 1:
                    ds_s[c - 1].wait()
                ds_g[c + 1] = start_g(c + 1)
            scale(c)
            ds_s[c] = start_s(c)
        ds_s[nch - 1].wait()

    return _gather_kernel(x2d, gidx, wsel)


def _scale_body(g_ref, w_ref, o_ref):
    o_ref[...] = g_ref[...] * w_ref[...]


def _scale_rows(gath, wcol, rows_blk=1024):
    n, h = gath.shape
    return pl.pallas_call(
        _scale_body,
        grid=(n // rows_blk,),
        in_specs=[
            pl.BlockSpec((rows_blk, h), lambda i: (i, 0)),
            pl.BlockSpec((rows_blk, 1), lambda i: (i, 0)),
        ],
        out_specs=pl.BlockSpec((rows_blk, h), lambda i: (i, 0)),
        out_shape=jax.ShapeDtypeStruct((n, h), jnp.float32),
        compiler_params=pltpu.CompilerParams(
            dimension_semantics=("parallel",)),
    )(gath, wcol)


def kernel(x, W_router):
    b, s, h = x.shape
    k = max(1, (s * 1) // 2)
    x2d = x.reshape(b * s, h)
    z2d, s2d = _compute_scores(x2d, W_router)
    t_out, n_out = _compute_threshold(z2d, b, s, k)
    gidx, wsel = _compact(z2d.reshape(b * s), s2d.reshape(b * s),
                          t_out.reshape(-1), n_out.reshape(-1), b, s, k)
    out2d = _gather(x2d, gidx, wsel, b, s, h, k)
    return out2d.reshape(b, k, h)


# speedup vs baseline: 2.3268x; 1.0042x over previous
import functools

import jax
import jax.numpy as jnp
from jax import lax
from jax.experimental import pallas as pl
from jax.experimental.pallas import tpu as pltpu
from jax.experimental.pallas import tpu_sc as plsc

_ALPHA = 0.1
_LANES = 16
_SC_PARAMS = pltpu.CompilerParams(needs_layout_passes=False)


def _scores_body(x_ref, w_ref, z_ref, s_ref):
    xb = x_ref[...].astype(jnp.bfloat16)
    wb = w_ref[...].astype(jnp.bfloat16)
    z = jax.lax.dot_general(xb, wb.T, (((1,), (0,)), ((), ())),
                            preferred_element_type=jnp.float32)
    z = z.reshape(z_ref.shape)
    z_ref[...] = z
    s_ref[...] = _ALPHA * (1.0 / (1.0 + jnp.exp(-z)))


def _compute_scores(x2d, w):
    n, h = x2d.shape
    rows = 1024
    return pl.pallas_call(
        _scores_body,
        grid=(n // rows,),
        in_specs=[
            pl.BlockSpec((rows, h), lambda i: (i, 0)),
            pl.BlockSpec((1, h), lambda i: (0, 0)),
        ],
        out_specs=[pl.BlockSpec((rows // 128, 128), lambda i: (i, 0))] * 2,
        out_shape=[jax.ShapeDtypeStruct((n // 128, 128), jnp.float32)] * 2,
        compiler_params=pltpu.CompilerParams(
            dimension_semantics=("parallel",)),
    )(x2d, w)


def _thresh_body(k, b, s, z_ref, t_ref, n_ref):
    z = z_ref[...].reshape(b, s)
    u = lax.bitcast_convert_type(z, jnp.int32)
    key = jnp.bitwise_xor(
        u, jnp.bitwise_and(jnp.right_shift(u, 31), jnp.int32(0x7FFFFFFF)))

    cnt_nn = jnp.sum((key >= 0).astype(jnp.int32), axis=1, keepdims=True)
    t0 = jnp.where(cnt_nn >= k, jnp.int32(0), jnp.int32(-2147483648))

    def step(i, t_acc):
        bit = 30 - i
        cand = jnp.bitwise_or(t_acc, jnp.left_shift(jnp.int32(1), bit))
        cnt = jnp.sum((key >= cand).astype(jnp.int32), axis=1, keepdims=True)
        return jnp.where(cnt >= k, cand, t_acc)

    tk = lax.fori_loop(0, 31, step, t0)
    uz = jnp.bitwise_xor(
        tk, jnp.bitwise_and(jnp.right_shift(tk, 31), jnp.int32(0x7FFFFFFF)))
    t_z = lax.bitcast_convert_type(uz, jnp.float32)
    n_gt = jnp.sum((z > t_z).astype(jnp.int32), axis=1, keepdims=True)
    need = k - n_gt
    t_ref[...] = jnp.broadcast_to(t_z, (b, _LANES))
    n_ref[...] = jnp.broadcast_to(need, (b, _LANES))


def _compute_threshold(z2d, b, s, k):
    return pl.pallas_call(
        functools.partial(_thresh_body, k, b, s),
        in_specs=[pl.BlockSpec(z2d.shape, lambda: (0, 0))],
        out_specs=[
            pl.BlockSpec((b, _LANES), lambda: (0, 0)),
            pl.BlockSpec((b, _LANES), lambda: (0, 0)),
        ],
        out_shape=[
            jax.ShapeDtypeStruct((b, _LANES), jnp.float32),
            jax.ShapeDtypeStruct((b, _LANES), jnp.int32),
        ],
    )(z2d)


def _compact(z_flat, s_flat, t_flat, n_flat, b, s, k):
    mesh = plsc.VectorSubcoreMesh(core_axis_name="c", subcore_axis_name="s",
                                  num_cores=2, num_subcores=16)

    @functools.partial(
        pl.kernel,
        out_type=(
            jax.ShapeDtypeStruct((b * k,), jnp.int32),
            jax.ShapeDtypeStruct((b * k,), jnp.float32),
        ),
        mesh=mesh,
        scratch_types=[
            pltpu.VMEM((s,), jnp.float32),
            pltpu.VMEM((s,), jnp.float32),
            pltpu.VMEM((_LANES,), jnp.float32),
            pltpu.VMEM((_LANES,), jnp.int32),
            pltpu.VMEM((k + _LANES,), jnp.int32),
            pltpu.VMEM((k + _LANES,), jnp.float32),
        ],
        compiler_params=_SC_PARAMS,
    )
    def _compact_kernel(z_hbm, s_hbm, t_hbm, n_hbm, gi_hbm, ws_hbm,
                        zv, sv, tv, nv, gbuf, wbuf):
        cid = lax.axis_index("c")
        sid = lax.axis_index("s")

        bpc = (b + 1) // 2

        @pl.when(jnp.logical_and(sid < bpc, cid * bpc + sid < b))
        def _():
            bb = cid * bpc + sid
            pltpu.sync_copy(z_hbm.at[pl.ds(bb * s, s)], zv)
            pltpu.sync_copy(s_hbm.at[pl.ds(bb * s, s)], sv)
            pltpu.sync_copy(t_hbm.at[pl.ds(bb * _LANES, _LANES)], tv)
            pltpu.sync_copy(n_hbm.at[pl.ds(bb * _LANES, _LANES)], nv)
            t = tv[...]
            nd = nv[...]
            base = bb * s

            def step(j, carry):
                off, eqc = carry
                zv16 = zv[pl.ds(j * _LANES, _LANES)]
                sv16 = sv[pl.ds(j * _LANES, _LANES)]
                idx = lax.iota(jnp.int32, _LANES) + (base + j * _LANES)
                gt = zv16 > t
                eqm = zv16 == t
                eqi = eqm.astype(jnp.int32)
                csum = plsc.cumsum(eqi)
                rank_before = (csum - eqi) + eqc
                sel = jnp.logical_or(gt, jnp.logical_and(eqm, rank_before < nd))
                plsc.store_compressed(gbuf.at[pl.ds(off, _LANES)], idx,
                                      mask=sel)
                plsc.store_compressed(wbuf.at[pl.ds(off, _LANES)], sv16,
                                      mask=sel)
                off = off + jnp.sum(sel.astype(jnp.int32))
                eqc = eqc + jnp.sum(eqi)
                return off, eqc

            lax.fori_loop(0, s // _LANES, step,
                          (jnp.int32(0), jnp.int32(0)))
            pltpu.sync_copy(gbuf.at[pl.ds(0, k)], gi_hbm.at[pl.ds(bb * k, k)])
            pltpu.sync_copy(wbuf.at[pl.ds(0, k)], ws_hbm.at[pl.ds(bb * k, k)])

    return _compact_kernel(z_flat, s_flat, t_flat, n_flat)


def _gather(x2d, gidx, wsel, b, s, h, k):
    mesh = plsc.VectorSubcoreMesh(core_axis_name="c", subcore_axis_name="s",
                                  num_cores=2, num_subcores=16)
    nw = 32
    rpw = (b * k) // nw
    ch = 16
    nch = rpw // ch

    @functools.partial(
        pl.kernel,
        out_type=jax.ShapeDtypeStruct((b * k, h), jnp.float32),
        mesh=mesh,
        scratch_types=[
            pltpu.VMEM((rpw,), jnp.int32),
            pltpu.VMEM((rpw,), jnp.float32),
            pltpu.VMEM((ch, h), jnp.float32),
            pltpu.VMEM((ch, h), jnp.float32),
            pltpu.SemaphoreType.DMA,
            pltpu.SemaphoreType.DMA,
            pltpu.SemaphoreType.DMA,
            pltpu.SemaphoreType.DMA,
        ],
        compiler_params=_SC_PARAMS,
    )
    def _gather_kernel(x_hbm, gi_hbm, ws_hbm, o_hbm, gi, wv, bufa, bufb,
                       gsa, gsb, ssa, ssb):
        cid = lax.axis_index("c")
        sid = lax.axis_index("s")
        wid = sid * 2 + cid
        row0 = wid * rpw
        pltpu.sync_copy(gi_hbm.at[pl.ds(row0, rpw)], gi)
        pltpu.sync_copy(ws_hbm.at[pl.ds(row0, rpw)], wv)
        bufs = ((bufa, gsa, ssa), (bufb, gsb, ssb))

        def start_g(c):
            buf, gs, _ = bufs[c % 2]
            return pltpu.async_copy(x_hbm.at[gi.at[pl.ds(c * ch, ch)]],
                                    buf, gs)

        def start_s(c):
            buf, _, ss = bufs[c % 2]
            return pltpu.async_copy(buf, o_hbm.at[pl.ds(row0 + c * ch, ch)],
                                    ss)

        def scale(c):
            buf = bufs[c % 2][0]

            @pl.loop(0, ch)
            def _(r):
                wr = plsc.load_gather(
                    wv, [jnp.full((_LANES,), c * ch + r, jnp.int32)])

                @pl.loop(0, h, step=_LANES, unroll=8)
                def _(l):
                    buf[r, pl.ds(l, _LANES)] = buf[r, pl.ds(l, _LANES)] * wr

        ds_g = [None] * nch
        ds_s = [None] * nch
        ds_g[0] = start_g(0)
        for c in range(nch):
            ds_g[c].wait()
            if c + 1 < nch:
                if c >= 1:
                    ds_s[c - 1].wait()
                ds_g[c + 1] = start_g(c + 1)
            scale(c)
            ds_s[c] = start_s(c)
        ds_s[nch - 1].wait()

    return _gather_kernel(x2d, gidx, wsel)


def _scale_body(g_ref, w_ref, o_ref):
    o_ref[...] = g_ref[...] * w_ref[...]


def _scale_rows(gath, wcol, rows_blk=1024):
    n, h = gath.shape
    return pl.pallas_call(
        _scale_body,
        grid=(n // rows_blk,),
        in_specs=[
            pl.BlockSpec((rows_blk, h), lambda i: (i, 0)),
            pl.BlockSpec((rows_blk, 1), lambda i: (i, 0)),
        ],
        out_specs=pl.BlockSpec((rows_blk, h), lambda i: (i, 0)),
        out_shape=jax.ShapeDtypeStruct((n, h), jnp.float32),
        compiler_params=pltpu.CompilerParams(
            dimension_semantics=("parallel",)),
    )(gath, wcol)


def kernel(x, W_router):
    b, s, h = x.shape
    k = max(1, (s * 1) // 2)
    x2d = x.reshape(b * s, h)
    z2d, s2d = _compute_scores(x2d, W_router)
    t_out, n_out = _compute_threshold(z2d, b, s, k)
    gidx, wsel = _compact(z2d.reshape(b * s), s2d.reshape(b * s),
                          t_out.reshape(-1), n_out.reshape(-1), b, s, k)
    out2d = _gather(x2d, gidx, wsel, b, s, h, k)
    return out2d.reshape(b, k, h)
